# Initial kernel scaffold; baseline (speedup 1.0000x reference)
import jax
import jax.numpy as jnp
from jax.experimental import pallas as pl


def kernel(table, ix, target):
    raise NotImplementedError("write your pallas kernel here")


# speedup vs baseline: 1.2939x; 1.2939x over previous
import functools

import jax
import jax.numpy as jnp
from jax import lax
from jax.experimental import pallas as pl
from jax.experimental.pallas import tpu as pltpu
from jax.experimental.pallas import tpu_sc as plsc

V = 1000
B = 4096
T = 50
N = B * T

_info = plsc.get_sparse_core_info()
NC = _info.num_cores
NS = _info.num_subcores
NW = NC * NS
PER_W = N // NW
CHUNK = 32
NCHUNK = PER_W // CHUNK


def _lse_body(table_ref, lse_ref):
    x = table_ref[...]
    m = jnp.max(x, axis=1, keepdims=True)
    s = jnp.sum(jnp.exp(x - m), axis=1, keepdims=True)
    lse_ref[...] = (m + jnp.log(s))[:, 0]


def _loss_body(part_ref, loss_ref):
    loss_ref[...] = jnp.sum(part_ref[...], keepdims=True).reshape(1, 1) / N


def _sc_body(table_hbm, ix_hbm, tgt_hbm, lse_hbm,
             out_hbm, part_hbm,
             idx_v, tgt_v, rows_v, lse_v, acc_v, gsem):
    wid = lax.axis_index("s") * NC + lax.axis_index("c")
    base = wid * PER_W
    pltpu.sync_copy(lse_hbm, lse_v)
    acc_v[...] = jnp.zeros((16,), jnp.float32)

    def chunk_body(c, carry):
        off = base + c * CHUNK
        pltpu.sync_copy(ix_hbm.at[pl.ds(off, CHUNK)], idx_v)
        pltpu.sync_copy(tgt_hbm.at[pl.ds(off, CHUNK)], tgt_v)
        pltpu.async_copy(table_hbm.at[idx_v], rows_v, gsem).wait()
        for j in range(CHUNK // 16):
            ixv = idx_v[pl.ds(j * 16, 16)]
            tg = tgt_v[pl.ds(j * 16, 16)]
            rid = lax.iota(jnp.int32, 16) + (j * 16)
            a = plsc.load_gather(lse_v, [ixv])
            b = plsc.load_gather(rows_v, [rid, tg])
            acc_v[...] = acc_v[...] + (a - b)
        pltpu.sync_copy(rows_v, out_hbm.at[pl.ds(off, CHUNK)])
        return carry

    lax.fori_loop(0, NCHUNK, chunk_body, 0)
    pltpu.sync_copy(acc_v, part_hbm.at[wid])


def kernel(table, ix, target):
    lse = pl.pallas_call(
        _lse_body,
        out_shape=jax.ShapeDtypeStruct((V,), jnp.float32),
    )(table)

    mesh = plsc.VectorSubcoreMesh(core_axis_name="c", subcore_axis_name="s")
    sc = pl.kernel(
        _sc_body,
        mesh=mesh,
        out_type=[
            jax.ShapeDtypeStruct((N, V), jnp.float32),
            jax.ShapeDtypeStruct((NW, 16), jnp.float32),
        ],
        scratch_types=[
            pltpu.VMEM((CHUNK,), jnp.int32),
            pltpu.VMEM((CHUNK,), jnp.int32),
            pltpu.VMEM((CHUNK, V), jnp.float32),
            pltpu.VMEM((V,), jnp.float32),
            pltpu.VMEM((16,), jnp.float32),
            pltpu.SemaphoreType.DMA,
        ],
        compiler_params=pltpu.CompilerParams(
            use_tc_tiling_on_sc=False, needs_layout_passes=False
        ),
    )
    flat_logits, partials = sc(table, ix.reshape(N), target.reshape(N), lse)

    loss2d = pl.pallas_call(
        _loss_body,
        out_shape=jax.ShapeDtypeStruct((1, 1), jnp.float32),
    )(partials)

    return flat_logits.reshape(B, T, V), loss2d[0, 0]


# speedup vs baseline: 1.4433x; 1.1154x over previous
import functools

import jax
import jax.numpy as jnp
from jax import lax
from jax.experimental import pallas as pl
from jax.experimental.pallas import tpu as pltpu
from jax.experimental.pallas import tpu_sc as plsc

V = 1000
B = 4096
T = 50
N = B * T

_info = plsc.get_sparse_core_info()
NC = _info.num_cores
NS = _info.num_subcores
NW = NC * NS
PER_W = N // NW
CHUNK = 32
NCHUNK = PER_W // CHUNK


def _lse_body(table_ref, lse_ref):
    x = table_ref[...]
    m = jnp.max(x, axis=1, keepdims=True)
    s = jnp.sum(jnp.exp(x - m), axis=1, keepdims=True)
    lse_ref[...] = (m + jnp.log(s))[:, 0]


def _loss_body(part_ref, loss_ref):
    loss_ref[...] = jnp.sum(part_ref[...], keepdims=True).reshape(1, 1) / N


def _sc_body(table_hbm, ix_hbm, tgt_hbm, lse_hbm,
             out_hbm, part_hbm,
             ixall_v, tgall_v, rows0_v, rows1_v, lse_v, acc_v,
             gsem0, gsem1, wsem0, wsem1):
    wid = lax.axis_index("s") * NC + lax.axis_index("c")
    base = wid * PER_W
    pltpu.sync_copy(ix_hbm.at[pl.ds(base, PER_W)], ixall_v)
    pltpu.sync_copy(tgt_hbm.at[pl.ds(base, PER_W)], tgall_v)
    pltpu.sync_copy(lse_hbm, lse_v)
    acc_v[...] = jnp.zeros((16,), jnp.float32)

    rows = (rows0_v, rows1_v)
    gsem = (gsem0, gsem1)
    wsem = (wsem0, wsem1)

    def g_copy(c, b):
        return pltpu.make_async_copy(
            table_hbm.at[ixall_v.at[pl.ds(c * CHUNK, CHUNK)]],
            rows[b], gsem[b])

    def w_copy(c, b):
        return pltpu.make_async_copy(
            rows[b], out_hbm.at[pl.ds(base + c * CHUNK, CHUNK)], wsem[b])

    g_copy(0, 0).start()

    def chunk_step(c, b):
        g_copy(c, b).wait()
        loc = c * CHUNK
        s = jnp.zeros((16,), jnp.float32)
        for j in range(CHUNK // 16):
            ixv = ixall_v[pl.ds(loc + j * 16, 16)]
            tg = tgall_v[pl.ds(loc + j * 16, 16)]
            rid = lax.iota(jnp.int32, 16) + (j * 16)
            a = plsc.load_gather(lse_v, [ixv])
            bb = plsc.load_gather(rows[b], [rid, tg])
            s = s + (a - bb)
        acc_v[...] = acc_v[...] + s
        w_copy(c, b).start()
        ob = 1 - b

        @pl.when(c + 1 < NCHUNK)
        def _():
            @pl.when(c >= 1)
            def _():
                w_copy(c - 1, ob).wait()
            g_copy(c + 1, ob).start()

    def body(g, carry):
        chunk_step(2 * g, 0)
        chunk_step(2 * g + 1, 1)
        return carry

    lax.fori_loop(0, NCHUNK // 2, body, 0)
    w_copy(NCHUNK - 2, 0).wait()
    w_copy(NCHUNK - 1, 1).wait()
    pltpu.sync_copy(acc_v, part_hbm.at[wid])


def kernel(table, ix, target):
    lse = pl.pallas_call(
        _lse_body,
        out_shape=jax.ShapeDtypeStruct((V,), jnp.float32),
    )(table)

    mesh = plsc.VectorSubcoreMesh(core_axis_name="c", subcore_axis_name="s")
    sc = pl.kernel(
        _sc_body,
        mesh=mesh,
        out_type=[
            jax.ShapeDtypeStruct((N, V), jnp.float32),
            jax.ShapeDtypeStruct((NW, 16), jnp.float32),
        ],
        scratch_types=[
            pltpu.VMEM((PER_W,), jnp.int32),
            pltpu.VMEM((PER_W,), jnp.int32),
            pltpu.VMEM((CHUNK, V), jnp.float32),
            pltpu.VMEM((CHUNK, V), jnp.float32),
            pltpu.VMEM((V,), jnp.float32),
            pltpu.VMEM((16,), jnp.float32),
            pltpu.SemaphoreType.DMA,
            pltpu.SemaphoreType.DMA,
            pltpu.SemaphoreType.DMA,
            pltpu.SemaphoreType.DMA,
        ],
        compiler_params=pltpu.CompilerParams(
            use_tc_tiling_on_sc=False, needs_layout_passes=False
        ),
    )
    flat_logits, partials = sc(table, ix.reshape(N), target.reshape(N), lse)

    loss2d = pl.pallas_call(
        _loss_body,
        out_shape=jax.ShapeDtypeStruct((1, 1), jnp.float32),
    )(partials)

    return flat_logits.reshape(B, T, V), loss2d[0, 0]
